# baseline (speedup 1.0000x reference)
import functools

import jax
import jax.numpy as jnp
from jax import lax
from jax.experimental import pallas as pl
from jax.experimental.pallas import tpu as pltpu
from jax.experimental.pallas import tpu_sc as plsc

NUM_CORES = 2
NUM_SUBCORES = 16
LANES = 16
G = 80
HALF = 32


@functools.lru_cache(maxsize=None)
def _build_sc_propagate(n_nodes: int, n_groups: int):
    gpt = n_groups // NUM_SUBCORES
    scs = 8
    for cand in range(32, 0, -8):
        if gpt % cand == 0:
            scs = cand
            break
    scn = gpt // scs
    rpt = n_nodes // NUM_SUBCORES
    zq, zr = divmod(rpt, G)
    nbuf = 8

    mesh = plsc.VectorSubcoreMesh(core_axis_name="c", subcore_axis_name="s")
    emb_sd = jax.ShapeDtypeStruct((NUM_CORES, n_nodes, HALF // 2), jnp.int32)
    nsb = 2

    @functools.partial(
        pl.kernel,
        out_type=(emb_sd, emb_sd, emb_sd),
        mesh=mesh,
        compiler_params=pltpu.CompilerParams(use_tc_tiling_on_sc=False,
                                             needs_layout_passes=False),
        scratch_types=[
            pltpu.VMEM_SHARED((n_nodes, HALF), jnp.float32),
            pltpu.VMEM((scs, G), jnp.int32),
            pltpu.VMEM((scs, G), jnp.int32),
            pltpu.VMEM((scs, G), jnp.float32),
        ] + [pltpu.VMEM((G, HALF // 2), jnp.int32)] * nbuf
          + [pltpu.VMEM((G, HALF), jnp.float32)] * nsb
          + [pltpu.VMEM((G, HALF), jnp.float32)]
          + [pltpu.VMEM((G, HALF // 2), jnp.int32)]
          + [pltpu.SemaphoreType.DMA] * (nbuf + nsb + 1),
    )
    def propagate(ego0, colr, rowr, valr, donor, l1, l2, l3,
                  acc, colb, rowb, valb, *rest):
        gr = rest[:nbuf]
        sb = rest[nbuf:nbuf + nsb]
        zbuf = rest[nbuf + nsb]
        obuf = rest[nbuf + nsb + 1]
        sg = rest[nbuf + nsb + 2:2 * nbuf + nsb + 2]
        ss = rest[2 * nbuf + nsb + 2:2 * nbuf + 2 * nsb + 2]
        sz = rest[2 * nbuf + 2 * nsb + 2]
        c = lax.axis_index("c")
        s = lax.axis_index("s")
        z16 = jnp.zeros((16,), jnp.float32)
        himask = jnp.int32(-65536)

        rowbase = s * rpt
        gbase = s * gpt
        srcs = (ego0, l1, l2)
        dsts = (l1, l2, l3)

        def scale(buf, out, jj):
            for sub in range(G // LANES):
                vv = valb[jj, pl.ds(sub * LANES, LANES)]
                for e in range(LANES):
                    idx = sub * LANES + e
                    v = vv[e]
                    w = buf[idx]
                    xa = plsc.bitcast(lax.shift_left(w, 16), jnp.float32)
                    xb = plsc.bitcast(jnp.bitwise_and(w, himask), jnp.float32)
                    out[idx, pl.ds(0, LANES)] = xa * v
                    out[idx, pl.ds(LANES, LANES)] = xb * v

        def zrow(i, carry):
            for h in range(HALF // LANES):
                zbuf[i, pl.ds(h * LANES, LANES)] = z16
            return carry
        lax.fori_loop(0, G, zrow, 0)

        for li in range(3):
            src = srcs[li]
            dst = dsts[li]
            dummy = src.at[c].at[pl.ds(0, G)]

            descs = [pltpu.async_copy(
                zbuf, acc.at[pl.ds(rowbase + k * G, G)], sz)
                for k in range(zq)]
            if zr:
                descs.append(pltpu.async_copy(
                    zbuf.at[pl.ds(0, zr)],
                    acc.at[pl.ds(rowbase + zq * G, zr)], sz))
            for d_ in descs:
                d_.wait()
            plsc.subcore_barrier()

            def superchunk(sc_i, carry):
                gb = gbase + sc_i * scs
                d1 = pltpu.async_copy(colr.at[pl.ds(gb, scs)], colb, sz)
                d2 = pltpu.async_copy(rowr.at[pl.ds(gb, scs)], rowb, sz)
                d3 = pltpu.async_copy(valr.at[pl.ds(gb, scs)], valb, sz)
                d1.wait(); d2.wait(); d3.wait()
                for p in range(4):
                    pltpu.async_copy(src.at[c].at[colb.at[p]], gr[p], sg[p])

                def ring(j0, carry2):
                    for b in range(nbuf):
                        jj = j0 * nbuf + b
                        b2 = (b + 4) % nbuf
                        sbi = b % nsb
                        pltpu.make_async_copy(dummy, gr[b], sg[b]).wait()

                        @pl.when(jj + 4 < scs)
                        def _():
                            pltpu.async_copy(src.at[c].at[colb.at[jj + 4]],
                                             gr[b2], sg[b2])

                        @pl.when(jj >= nsb)
                        def _():
                            pltpu.make_async_copy(donor, sb[sbi],
                                                  ss[sbi]).wait()
                        scale(gr[b], sb[sbi], jj)
                        pltpu.async_copy(sb[sbi], acc.at[rowb.at[jj]],
                                         ss[sbi], add=True)
                    return carry2
                lax.fori_loop(0, scs // nbuf, ring, 0)
                for jj in range(scs - nsb, scs):
                    pltpu.make_async_copy(donor, sb[jj % nsb],
                                          ss[jj % nsb]).wait()
                return carry
            lax.fori_loop(0, scn, superchunk, 0)
            plsc.subcore_barrier()

            def chunk_out(base, nrow):
                pltpu.sync_copy(acc.at[pl.ds(base, nrow)],
                                zbuf.at[pl.ds(0, nrow)])
                for i in range(nrow):
                    wa = plsc.bitcast(zbuf[i, pl.ds(0, LANES)], jnp.int32)
                    wb = plsc.bitcast(zbuf[i, pl.ds(LANES, LANES)], jnp.int32)
                    obuf[i] = jnp.bitwise_or(lax.shift_right_logical(wa, 16),
                                             jnp.bitwise_and(wb, himask))
                pltpu.sync_copy(obuf.at[pl.ds(0, nrow)],
                                dst.at[c].at[pl.ds(base, nrow)])

            def cloop(k, carry):
                chunk_out(rowbase + k * G, G)
                return carry
            lax.fori_loop(0, zq, cloop, 0)
            if zr:
                chunk_out(rowbase + zq * G, zr)
            lax.fori_loop(0, G, zrow, 0)
            plsc.subcore_barrier()

    return propagate


@functools.lru_cache(maxsize=None)
def _build_mean4(total: int):
    rows = total // 128
    blk = rows
    for cand in range(1024, 0, -16):
        if rows % cand == 0:
            blk = cand
            break
    grid = rows // blk

    def mean_body(a, b, c, d, o):
        o[...] = (a[...] + (b[...].astype(jnp.float32) +
                            c[...].astype(jnp.float32) +
                            d[...].astype(jnp.float32))) * 0.25

    spec = pl.BlockSpec((blk, 128), lambda i: (i, 0))
    call = pl.pallas_call(
        mean_body,
        out_shape=jax.ShapeDtypeStruct((rows, 128), jnp.float32),
        grid=(grid,),
        in_specs=[spec] * 4,
        out_specs=spec,
    )

    def mean4(a, b, c, d):
        r = lambda x: x.reshape(rows, 128)
        return call(r(a), r(b), r(c), r(d)).reshape(a.shape)
    return mean4


def kernel(user_emb, item_emb, adj_indices, adj_values):
    nu = user_emb.shape[0]
    n = nu + item_emb.shape[0]
    d = user_emb.shape[1]
    e = adj_values.shape[0]

    n_pad = -(-n // 128) * 128
    groups = -(-e // G)
    groups_pad = -(-groups // 2048) * 2048
    e_pad = groups_pad * G

    ego0 = jnp.concatenate([user_emb, item_emb], axis=0)
    ego0_p = jnp.zeros((n_pad, d), jnp.float32).at[:n].set(ego0)
    ego0_st = ego0_p.reshape(n_pad, 2, HALF).transpose(1, 0, 2)

    idx32 = adj_indices.astype(jnp.int32)
    zi = jnp.zeros((e_pad - e,), jnp.int32)
    row = jnp.concatenate([idx32[0], zi]).reshape(groups_pad, G)
    col = jnp.concatenate([idx32[1], zi]).reshape(groups_pad, G)
    vals = jnp.concatenate(
        [adj_values, jnp.zeros((e_pad - e,), jnp.float32)]).reshape(
            groups_pad, G)

    ego0_i = lax.bitcast_convert_type(
        ego0_st.astype(jnp.bfloat16).reshape(2, n_pad, HALF // 2, 2),
        jnp.int32)
    donor = jnp.zeros((G, HALF), jnp.float32)
    l1, l2, l3 = _build_sc_propagate(n_pad, groups_pad)(
        ego0_i, col, row, vals, donor)
    unbc = lambda x: lax.bitcast_convert_type(
        x, jnp.bfloat16).reshape(2, n_pad, HALF)
    l1, l2, l3 = unbc(l1), unbc(l2), unbc(l3)
    final_st = _build_mean4(2 * n_pad * HALF)(ego0_st, l1, l2, l3)
    final = final_st.transpose(1, 0, 2).reshape(n_pad, d)
    return final[:nu], final[nu:n]


# speedup vs baseline: 3.0314x; 3.0314x over previous
import functools

import jax
import jax.numpy as jnp
from jax import lax
from jax.experimental import pallas as pl
from jax.experimental.pallas import tpu as pltpu
from jax.experimental.pallas import tpu_sc as plsc

NUM_CORES = 2
NUM_SUBCORES = 16
LANES = 16
G = 80
HALF = 32


@functools.lru_cache(maxsize=None)
def _build_sc_propagate(n_nodes: int, n_groups: int):
    gpt = n_groups // NUM_SUBCORES
    scs = 8
    for cand in range(32, 0, -8):
        if gpt % cand == 0:
            scs = cand
            break
    scn = gpt // scs
    rpt = n_nodes // NUM_SUBCORES
    zq, zr = divmod(rpt, G)
    nbuf = 8

    mesh = plsc.VectorSubcoreMesh(core_axis_name="c", subcore_axis_name="s")
    emb_sd = jax.ShapeDtypeStruct((NUM_CORES, n_nodes, HALF), jnp.float32)

    @functools.partial(
        pl.kernel,
        out_type=(emb_sd, emb_sd, emb_sd),
        mesh=mesh,
        compiler_params=pltpu.CompilerParams(use_tc_tiling_on_sc=False),
        scratch_types=[
            pltpu.VMEM_SHARED((n_nodes, HALF), jnp.float32),
            pltpu.VMEM((scs, G), jnp.int32),
            pltpu.VMEM((scs, G), jnp.int32),
            pltpu.VMEM((scs, G), jnp.float32),
        ] + [pltpu.VMEM((G, HALF), jnp.float32)] * nbuf
          + [pltpu.SemaphoreType.DMA] * (2 * nbuf + 1),
    )
    def propagate(ego0, colr, rowr, valr, l1, l2, l3,
                  acc, colb, rowb, valb, *rest):
        gr = rest[:nbuf]
        sg = rest[nbuf:2 * nbuf]
        ss = rest[2 * nbuf:3 * nbuf]
        sz = rest[3 * nbuf]
        c = lax.axis_index("c")
        s = lax.axis_index("s")
        z16 = jnp.zeros((16,), jnp.float32)

        rowbase = s * rpt
        gbase = s * gpt
        srcs = (ego0, l1, l2)
        dsts = (l1, l2, l3)

        def scale(buf, jj):
            for sub in range(G // LANES):
                vv = valb[jj, pl.ds(sub * LANES, LANES)]
                for e in range(LANES):
                    idx = sub * LANES + e
                    v = vv[e]
                    for h in range(HALF // LANES):
                        sl = pl.ds(h * LANES, LANES)
                        buf[idx, sl] = buf[idx, sl] * v

        for li in range(3):
            src = srcs[li]
            dst = dsts[li]
            dummy = src.at[c].at[pl.ds(0, G)]

            def zrow(i, carry):
                for h in range(HALF // LANES):
                    gr[0][i, pl.ds(h * LANES, LANES)] = z16
                return carry
            lax.fori_loop(0, G, zrow, 0)
            descs = [pltpu.async_copy(
                gr[0], acc.at[pl.ds(rowbase + k * G, G)], sz)
                for k in range(zq)]
            if zr:
                descs.append(pltpu.async_copy(
                    gr[0].at[pl.ds(0, zr)],
                    acc.at[pl.ds(rowbase + zq * G, zr)], sz))
            for d_ in descs:
                d_.wait()
            plsc.subcore_barrier()

            def superchunk(sc_i, carry):
                gb = gbase + sc_i * scs
                d1 = pltpu.async_copy(colr.at[pl.ds(gb, scs)], colb, sz)
                d2 = pltpu.async_copy(rowr.at[pl.ds(gb, scs)], rowb, sz)
                d3 = pltpu.async_copy(valr.at[pl.ds(gb, scs)], valb, sz)
                d1.wait(); d2.wait(); d3.wait()
                for p in range(6):
                    pltpu.async_copy(src.at[c].at[colb.at[p]], gr[p], sg[p])

                def ring(j0, carry2):
                    for b in range(nbuf):
                        jj = j0 * nbuf + b
                        b2 = (b + 6) % nbuf
                        pltpu.make_async_copy(dummy, gr[b], sg[b]).wait()
                        scale(gr[b], jj)
                        pltpu.async_copy(gr[b], acc.at[rowb.at[jj]], ss[b],
                                         add=True)

                        @pl.when(jj >= 2)
                        def _():
                            pltpu.make_async_copy(dummy, gr[b2], ss[b2]).wait()

                        @pl.when(jj + 6 < scs)
                        def _():
                            pltpu.async_copy(src.at[c].at[colb.at[jj + 6]],
                                             gr[b2], sg[b2])
                    return carry2
                lax.fori_loop(0, scs // nbuf, ring, 0)
                for jj in range(scs - 2, scs):
                    pltpu.make_async_copy(dummy, gr[jj % nbuf],
                                          ss[jj % nbuf]).wait()
                return carry
            lax.fori_loop(0, scn, superchunk, 0)
            plsc.subcore_barrier()
            sl = pl.ds(rowbase, rpt)
            pltpu.sync_copy(acc.at[sl], dst.at[c].at[sl])

    return propagate


@functools.lru_cache(maxsize=None)
def _build_mean4(total: int):
    rows = total // 128
    blk = rows
    for cand in range(1024, 0, -8):
        if rows % cand == 0:
            blk = cand
            break
    grid = rows // blk

    def mean_body(a, b, c, d, o):
        o[...] = (a[...] + b[...] + c[...] + d[...]) * 0.25

    spec = pl.BlockSpec((blk, 128), lambda i: (i, 0))
    call = pl.pallas_call(
        mean_body,
        out_shape=jax.ShapeDtypeStruct((rows, 128), jnp.float32),
        grid=(grid,),
        in_specs=[spec] * 4,
        out_specs=spec,
    )

    def mean4(a, b, c, d):
        r = lambda x: x.reshape(rows, 128)
        return call(r(a), r(b), r(c), r(d)).reshape(a.shape)
    return mean4


def kernel(user_emb, item_emb, adj_indices, adj_values):
    nu = user_emb.shape[0]
    n = nu + item_emb.shape[0]
    d = user_emb.shape[1]
    e = adj_values.shape[0]

    n_pad = -(-n // 128) * 128
    groups = -(-e // G)
    groups_pad = -(-groups // 2048) * 2048
    e_pad = groups_pad * G

    ego0 = jnp.concatenate([user_emb, item_emb], axis=0)
    ego0_p = jnp.zeros((n_pad, d), jnp.float32).at[:n].set(ego0)
    ego0_st = ego0_p.reshape(n_pad, 2, HALF).transpose(1, 0, 2)

    idx32 = adj_indices.astype(jnp.int32)
    zi = jnp.zeros((e_pad - e,), jnp.int32)
    row = jnp.concatenate([idx32[0], zi]).reshape(groups_pad, G)
    col = jnp.concatenate([idx32[1], zi]).reshape(groups_pad, G)
    vals = jnp.concatenate(
        [adj_values, jnp.zeros((e_pad - e,), jnp.float32)]).reshape(
            groups_pad, G)

    l1, l2, l3 = _build_sc_propagate(n_pad, groups_pad)(ego0_st, col, row, vals)
    final_st = _build_mean4(2 * n_pad * HALF)(ego0_st, l1, l2, l3)
    final = final_st.transpose(1, 0, 2).reshape(n_pad, d)
    return final[:nu], final[nu:n]


# speedup vs baseline: 3.2972x; 1.0877x over previous
import functools

import jax
import jax.numpy as jnp
from jax import lax
from jax.experimental import pallas as pl
from jax.experimental.pallas import tpu as pltpu
from jax.experimental.pallas import tpu_sc as plsc

NUM_CORES = 2
NUM_SUBCORES = 16
LANES = 16
G = 80
HALF = 32


@functools.lru_cache(maxsize=None)
def _build_sc_propagate(n_nodes: int, n_groups: int):
    gpt = n_groups // NUM_SUBCORES
    scs = 16
    scn = gpt // scs
    rpt = n_nodes // NUM_SUBCORES
    zq, zr = divmod(rpt, G)
    nbuf = 8

    mesh = plsc.VectorSubcoreMesh(core_axis_name="c", subcore_axis_name="s")
    emb_sd = jax.ShapeDtypeStruct((NUM_CORES, n_nodes, HALF), jnp.float32)

    @functools.partial(
        pl.kernel,
        out_type=(emb_sd, emb_sd, emb_sd),
        mesh=mesh,
        compiler_params=pltpu.CompilerParams(use_tc_tiling_on_sc=False),
        scratch_types=[
            pltpu.VMEM_SHARED((n_nodes, HALF), jnp.float32),
            pltpu.VMEM((2, scs, G), jnp.int32),
            pltpu.VMEM((2, scs, G), jnp.int32),
            pltpu.VMEM((2, scs, G), jnp.float32),
        ] + [pltpu.VMEM((G, HALF), jnp.float32)] * nbuf
          + [pltpu.SemaphoreType.DMA] * (2 * nbuf + 1),
    )
    def propagate(ego0, colr, rowr, valr, l1, l2, l3,
                  acc, colb, rowb, valb, *rest):
        gr = rest[:nbuf]
        sg = rest[nbuf:2 * nbuf]
        ss = rest[2 * nbuf:3 * nbuf]
        sz = rest[3 * nbuf]
        c = lax.axis_index("c")
        s = lax.axis_index("s")
        z16 = jnp.zeros((16,), jnp.float32)

        rowbase = s * rpt
        gbase = s * gpt
        srcs = (ego0, l1, l2)
        dsts = (l1, l2, l3)

        def scale(buf, par, loc):
            for sub in range(G // LANES):
                vv = valb[par, loc, pl.ds(sub * LANES, LANES)]
                for e in range(LANES):
                    idx = sub * LANES + e
                    v = vv[e]
                    for h in range(HALF // LANES):
                        sl = pl.ds(h * LANES, LANES)
                        buf[idx, sl] = buf[idx, sl] * v

        for li in range(3):
            src = srcs[li]
            dst = dsts[li]
            dummy = src.at[c].at[pl.ds(0, G)]

            def zrow(i, carry):
                for h in range(HALF // LANES):
                    gr[0][i, pl.ds(h * LANES, LANES)] = z16
                return carry
            lax.fori_loop(0, G, zrow, 0)
            descs = [pltpu.async_copy(
                gr[0], acc.at[pl.ds(rowbase + k * G, G)], sz)
                for k in range(zq)]
            if zr:
                descs.append(pltpu.async_copy(
                    gr[0].at[pl.ds(0, zr)],
                    acc.at[pl.ds(rowbase + zq * G, zr)], sz))
            for d_ in descs:
                d_.wait()
            plsc.subcore_barrier()

            def ldidx(k, par):
                gb = gbase + k * scs
                return [pltpu.async_copy(colr.at[pl.ds(gb, scs)],
                                         colb.at[par], sz),
                        pltpu.async_copy(rowr.at[pl.ds(gb, scs)],
                                         rowb.at[par], sz),
                        pltpu.async_copy(valr.at[pl.ds(gb, scs)],
                                         valb.at[par], sz)]

            def widx():
                for r_ in (colb, rowb, valb):
                    pltpu.make_async_copy(colr.at[pl.ds(0, scs)],
                                          r_.at[0], sz).wait()

            for d_ in ldidx(0, 0):
                d_.wait()
            for p in range(6):
                pltpu.async_copy(src.at[c].at[colb.at[0].at[p]],
                                 gr[p], sg[p])

            def ring(j0, carry2):
                for b in range(nbuf):
                    jj = j0 * nbuf + b
                    b2 = (b + 6) % nbuf
                    if b == 0:
                        sc_i = j0 // 2

                        @pl.when(jnp.logical_and(j0 % 2 == 0,
                                                 sc_i + 1 < scn))
                        def _():
                            ldidx(sc_i + 1, (sc_i + 1) % 2)

                        @pl.when(jnp.logical_and(j0 % 2 == 1,
                                                 sc_i + 1 < scn))
                        def _():
                            widx()
                    pltpu.make_async_copy(dummy, gr[b], sg[b]).wait()
                    scale(gr[b], (jj // scs) % 2, jj % scs)
                    pltpu.async_copy(
                        gr[b], acc.at[rowb.at[(jj // scs) % 2].at[jj % scs]],
                        ss[b], add=True)

                    @pl.when(jj >= 2)
                    def _():
                        pltpu.make_async_copy(dummy, gr[b2], ss[b2]).wait()

                    g2 = jj + 6

                    @pl.when(g2 < gpt)
                    def _():
                        pltpu.async_copy(
                            src.at[c].at[
                                colb.at[(g2 // scs) % 2].at[g2 % scs]],
                            gr[b2], sg[b2])
                return carry2
            lax.fori_loop(0, gpt // nbuf, ring, 0)
            for jj in range(gpt - 2, gpt):
                pltpu.make_async_copy(dummy, gr[jj % nbuf],
                                      ss[jj % nbuf]).wait()
            plsc.subcore_barrier()
            sl = pl.ds(rowbase, rpt)
            pltpu.sync_copy(acc.at[sl], dst.at[c].at[sl])

    return propagate


@functools.lru_cache(maxsize=None)
def _build_mean4(total: int):
    rows = total // 128
    blk = rows
    for cand in range(1024, 0, -8):
        if rows % cand == 0:
            blk = cand
            break
    grid = rows // blk

    def mean_body(a, b, c, d, o):
        o[...] = (a[...] + b[...] + c[...] + d[...]) * 0.25

    spec = pl.BlockSpec((blk, 128), lambda i: (i, 0))
    call = pl.pallas_call(
        mean_body,
        out_shape=jax.ShapeDtypeStruct((rows, 128), jnp.float32),
        grid=(grid,),
        in_specs=[spec] * 4,
        out_specs=spec,
    )

    def mean4(a, b, c, d):
        r = lambda x: x.reshape(rows, 128)
        return call(r(a), r(b), r(c), r(d)).reshape(a.shape)
    return mean4


def kernel(user_emb, item_emb, adj_indices, adj_values):
    nu = user_emb.shape[0]
    n = nu + item_emb.shape[0]
    d = user_emb.shape[1]
    e = adj_values.shape[0]

    n_pad = -(-n // 128) * 128
    groups = -(-e // G)
    groups_pad = -(-groups // 2048) * 2048
    e_pad = groups_pad * G

    ego0 = jnp.concatenate([user_emb, item_emb], axis=0)
    ego0_p = jnp.zeros((n_pad, d), jnp.float32).at[:n].set(ego0)
    ego0_st = ego0_p.reshape(n_pad, 2, HALF).transpose(1, 0, 2)

    idx32 = adj_indices.astype(jnp.int32)
    zi = jnp.zeros((e_pad - e,), jnp.int32)
    row = jnp.concatenate([idx32[0], zi]).reshape(groups_pad, G)
    col = jnp.concatenate([idx32[1], zi]).reshape(groups_pad, G)
    vals = jnp.concatenate(
        [adj_values, jnp.zeros((e_pad - e,), jnp.float32)]).reshape(
            groups_pad, G)

    l1, l2, l3 = _build_sc_propagate(n_pad, groups_pad)(ego0_st, col, row, vals)
    final_st = _build_mean4(2 * n_pad * HALF)(ego0_st, l1, l2, l3)
    final = final_st.transpose(1, 0, 2).reshape(n_pad, d)
    return final[:nu], final[nu:n]
